# Initial kernel scaffold; baseline (speedup 1.0000x reference)
import jax
import jax.numpy as jnp
from jax.experimental import pallas as pl


def kernel(feature, edge_index, W, b):
    raise NotImplementedError("write your pallas kernel here")


# speedup vs baseline: 5.5681x; 5.5681x over previous
import functools

import jax
import jax.numpy as jnp
from jax import lax
from jax.experimental import pallas as pl
from jax.experimental.pallas import tpu as pltpu
from jax.experimental.pallas import tpu_sc as plsc

N_NODES_C = 10000
N_EDGES_C = 320000
D = 128

NC = 2
NS = 16
NW = NC * NS

CHUNK = 80
PER_W = N_EDGES_C // NW
N_ITERS = PER_W // CHUNK

ROWS_MAIN = 624
ROWS_LAST = 640


def _sc_body(feat_hbm, src_hbm, dst_hbm, out_hbm,
             src_v, dst_v, rows_v, acc, sem):
    c = lax.axis_index("c")
    s = lax.axis_index("s")
    wid = s * NC + c

    zeros16 = jnp.zeros((16,), jnp.float32)

    def zrow(i, _):
        for j in range(D // 16):
            rows_v[i, pl.ds(j * 16, 16)] = zeros16
        return 0

    lax.fori_loop(0, CHUNK, zrow, 0, unroll=False)

    def zcopy(j, _):
        pltpu.sync_copy(rows_v, acc.at[pl.ds(s * ROWS_MAIN + j * CHUNK, CHUNK)])
        return 0

    lax.fori_loop(0, ROWS_MAIN // CHUNK, zcopy, 0, unroll=False)

    @pl.when(s < NS - 1)
    def _():
        rem = ROWS_MAIN - (ROWS_MAIN // CHUNK) * CHUNK
        pltpu.sync_copy(
            rows_v.at[pl.ds(0, rem)],
            acc.at[pl.ds(s * ROWS_MAIN + ROWS_MAIN - rem, rem)])

    @pl.when(s == NS - 1)
    def _():
        pltpu.sync_copy(
            rows_v,
            acc.at[pl.ds((NS - 1) * ROWS_MAIN + (ROWS_MAIN // CHUNK) * CHUNK,
                         ROWS_LAST - (ROWS_MAIN // CHUNK) * CHUNK)])

    plsc.subcore_barrier()

    base = wid * PER_W

    def body(i, _):
        off = base + i * CHUNK
        pltpu.sync_copy(src_hbm.at[pl.ds(off, CHUNK)], src_v)
        pltpu.sync_copy(dst_hbm.at[pl.ds(off, CHUNK)], dst_v)
        pltpu.async_copy(feat_hbm.at[src_v], rows_v, sem).wait()
        pltpu.sync_copy(rows_v, acc.at[dst_v], add=True)
        return 0

    lax.fori_loop(0, N_ITERS, body, 0, unroll=False)
    plsc.subcore_barrier()

    @pl.when(s < NS - 1)
    def _():
        pltpu.sync_copy(acc.at[pl.ds(s * ROWS_MAIN, ROWS_MAIN)],
                        out_hbm.at[pl.ds(c * N_NODES_C + s * ROWS_MAIN, ROWS_MAIN)])

    @pl.when(s == NS - 1)
    def _():
        pltpu.sync_copy(
            acc.at[pl.ds((NS - 1) * ROWS_MAIN, ROWS_LAST)],
            out_hbm.at[pl.ds(c * N_NODES_C + (NS - 1) * ROWS_MAIN, ROWS_LAST)])


@jax.jit
def _sc_aggregate(feature, src, dst):
    mesh = plsc.VectorSubcoreMesh(core_axis_name="c", subcore_axis_name="s")
    f = pl.kernel(
        _sc_body,
        out_type=jax.ShapeDtypeStruct((NC * N_NODES_C, D), jnp.float32),
        mesh=mesh,
        scratch_types=[
            pltpu.VMEM((CHUNK,), jnp.int32),
            pltpu.VMEM((CHUNK,), jnp.int32),
            pltpu.VMEM((CHUNK, D), jnp.float32),
            pltpu.VMEM_SHARED((N_NODES_C, D), jnp.float32),
            pltpu.SemaphoreType.DMA,
        ],
    )
    return f(feature, src, dst)


def _tc_body(p0_ref, p1_ref, wt_ref, b_ref, o_ref):
    agg = p0_ref[...] + p1_ref[...]
    h = jnp.dot(agg, wt_ref[...], preferred_element_type=jnp.float32)
    o_ref[...] = jnp.maximum(h + b_ref[...], 0.0)


@jax.jit
def _tc_update(partials, Wt, b2):
    blk = 1000
    grid = N_NODES_C // blk
    return pl.pallas_call(
        _tc_body,
        grid=(grid,),
        in_specs=[
            pl.BlockSpec((blk, D), lambda i: (i, 0)),
            pl.BlockSpec((blk, D), lambda i: (i + grid, 0)),
            pl.BlockSpec((D, D), lambda i: (0, 0)),
            pl.BlockSpec((1, D), lambda i: (0, 0)),
        ],
        out_specs=pl.BlockSpec((blk, D), lambda i: (i, 0)),
        out_shape=jax.ShapeDtypeStruct((N_NODES_C, D), jnp.float32),
    )(partials, partials, Wt, b2)


def kernel(feature, edge_index, W, b):
    src = edge_index[0]
    dst = edge_index[1]
    partials = _sc_aggregate(feature, src, dst)
    return _tc_update(partials, W.T, b.reshape(1, D))


# speedup vs baseline: 12.2618x; 2.2021x over previous
import jax
import jax.numpy as jnp
from jax import lax
from jax.experimental import pallas as pl
from jax.experimental.pallas import tpu as pltpu
from jax.experimental.pallas import tpu_sc as plsc

N_NODES_C = 10000
N_EDGES_C = 320000
D = 128

NC = 2
NS = 16
NW = NC * NS

CHUNK = 80
PER_W = N_EDGES_C // NW
N_ITERS = PER_W // CHUNK
N_PAIRS = (N_ITERS - 1) // 2

ROWS_MAIN = 624
ROWS_LAST = 640
ZSTEP = 80


def _sc_body(feat_hbm, src_hbm, dst_hbm, out_hbm,
             src_all, dst_all, rows0, rows1, acc, sem0, sem1, isem):
    c = lax.axis_index("c")
    s = lax.axis_index("s")
    wid = s * NC + c

    i0 = pltpu.async_copy(src_hbm.at[wid], src_all, isem)
    i1 = pltpu.async_copy(dst_hbm.at[wid], dst_all, isem)

    zeros16 = jnp.zeros((16,), jnp.float32)

    def zrow(i, _):
        for j in range(D // 16):
            rows0[i, pl.ds(j * 16, 16)] = zeros16
        return 0

    lax.fori_loop(0, CHUNK, zrow, 0, unroll=False)

    def zcopy(j, _):
        pltpu.sync_copy(rows0.at[pl.ds(0, ZSTEP)],
                        acc.at[pl.ds(s * ROWS_MAIN + j * ZSTEP, ZSTEP)])
        return 0

    lax.fori_loop(0, ROWS_MAIN // ZSTEP, zcopy, 0, unroll=False)

    @pl.when(s < NS - 1)
    def _():
        rem = ROWS_MAIN - (ROWS_MAIN // ZSTEP) * ZSTEP
        pltpu.sync_copy(
            rows0.at[pl.ds(0, rem)],
            acc.at[pl.ds(s * ROWS_MAIN + ROWS_MAIN - rem, rem)])

    @pl.when(s == NS - 1)
    def _():
        pltpu.sync_copy(
            rows0.at[pl.ds(0, ZSTEP)],
            acc.at[pl.ds((NS - 1) * ROWS_MAIN + (ROWS_MAIN // ZSTEP) * ZSTEP,
                         ZSTEP)])

    i0.wait()
    i1.wait()
    plsc.subcore_barrier()

    def sidx(k):
        return src_all.at[pl.ds(k * CHUNK, CHUNK)]

    pltpu.async_copy(feat_hbm.at[sidx(0)], rows0, sem0)

    def body(i, _):
        pltpu.async_copy(feat_hbm.at[sidx(2 * i + 1)], rows1, sem1)
        pltpu.make_async_copy(feat_hbm.at[sidx(2 * i)], rows0, sem0).wait()
        pltpu.sync_copy(rows0, acc.at[dst_all.at[2 * i]], add=True)
        pltpu.async_copy(feat_hbm.at[sidx(2 * i + 2)], rows0, sem0)
        pltpu.make_async_copy(feat_hbm.at[sidx(2 * i + 1)], rows1, sem1).wait()
        pltpu.sync_copy(rows1, acc.at[dst_all.at[2 * i + 1]], add=True)
        return 0

    lax.fori_loop(0, N_PAIRS, body, 0, unroll=False)
    pltpu.make_async_copy(feat_hbm.at[sidx(N_ITERS - 1)], rows0, sem0).wait()
    pltpu.sync_copy(rows0, acc.at[dst_all.at[N_ITERS - 1]], add=True)
    plsc.subcore_barrier()

    @pl.when(s < NS - 1)
    def _():
        pltpu.sync_copy(acc.at[pl.ds(s * ROWS_MAIN, ROWS_MAIN)],
                        out_hbm.at[pl.ds(c * N_NODES_C + s * ROWS_MAIN, ROWS_MAIN)])

    @pl.when(s == NS - 1)
    def _():
        pltpu.sync_copy(
            acc.at[pl.ds((NS - 1) * ROWS_MAIN, ROWS_LAST)],
            out_hbm.at[pl.ds(c * N_NODES_C + (NS - 1) * ROWS_MAIN, ROWS_LAST)])


@jax.jit
def _sc_aggregate(feature, src, dst):
    mesh = plsc.VectorSubcoreMesh(core_axis_name="c", subcore_axis_name="s")
    f = pl.kernel(
        _sc_body,
        out_type=jax.ShapeDtypeStruct((NC * N_NODES_C, D), jnp.float32),
        mesh=mesh,
        scratch_types=[
            pltpu.VMEM((PER_W,), jnp.int32),
            pltpu.VMEM((N_ITERS, CHUNK), jnp.int32),
            pltpu.VMEM((CHUNK, D), jnp.float32),
            pltpu.VMEM((CHUNK, D), jnp.float32),
            pltpu.VMEM_SHARED((N_NODES_C, D), jnp.float32),
            pltpu.SemaphoreType.DMA,
            pltpu.SemaphoreType.DMA,
            pltpu.SemaphoreType.DMA,
        ],
    )
    return f(feature, src, dst)


def _tc_body(p0_ref, p1_ref, wt_ref, b_ref, o_ref):
    agg = p0_ref[...] + p1_ref[...]
    h = jnp.dot(agg, wt_ref[...], preferred_element_type=jnp.float32)
    o_ref[...] = jnp.maximum(h + b_ref[...], 0.0)


@jax.jit
def _tc_update(partials, Wt, b2):
    blk = 1000
    grid = N_NODES_C // blk
    return pl.pallas_call(
        _tc_body,
        grid=(grid,),
        in_specs=[
            pl.BlockSpec((blk, D), lambda i: (i, 0)),
            pl.BlockSpec((blk, D), lambda i: (i + grid, 0)),
            pl.BlockSpec((D, D), lambda i: (0, 0)),
            pl.BlockSpec((1, D), lambda i: (0, 0)),
        ],
        out_specs=pl.BlockSpec((blk, D), lambda i: (i, 0)),
        out_shape=jax.ShapeDtypeStruct((N_NODES_C, D), jnp.float32),
    )(partials, partials, Wt, b2)


def kernel(feature, edge_index, W, b):
    src = edge_index[0].reshape(NW, PER_W)
    dst = edge_index[1].reshape(NW, N_ITERS, CHUNK)
    partials = _sc_aggregate(feature, src, dst)
    return _tc_update(partials, W.T, b.reshape(1, D))


# speedup vs baseline: 12.6953x; 1.0354x over previous
import jax
import jax.numpy as jnp
from jax import lax
from jax.experimental import pallas as pl
from jax.experimental.pallas import tpu as pltpu
from jax.experimental.pallas import tpu_sc as plsc

N_NODES_C = 10000
N_EDGES_C = 320000
D = 128

NC = 2
NS = 16
NW = NC * NS

CHUNK = 80
PER_W = N_EDGES_C // NW
N_ITERS = PER_W // CHUNK
N_PAIRS = (N_ITERS - 1) // 2

ROWS_MAIN = 624
ROWS_LAST = 640
ZSTEP = 80


def _sc_body(feat_hbm, e2_hbm, e4_hbm, out_hbm,
             src_all, dst_all, rows0, rows1, acc, sem0, sem1, isem):
    c = lax.axis_index("c")
    s = lax.axis_index("s")
    wid = s * NC + c

    i0 = pltpu.async_copy(e2_hbm.at[0, wid], src_all, isem)
    i1 = pltpu.async_copy(e4_hbm.at[1, wid], dst_all, isem)

    zeros16 = jnp.zeros((16,), jnp.float32)

    def zrow(i, _):
        for j in range(D // 16):
            rows0[i, pl.ds(j * 16, 16)] = zeros16
        return 0

    lax.fori_loop(0, CHUNK, zrow, 0, unroll=False)

    def zcopy(j, _):
        pltpu.sync_copy(rows0.at[pl.ds(0, ZSTEP)],
                        acc.at[pl.ds(s * ROWS_MAIN + j * ZSTEP, ZSTEP)])
        return 0

    lax.fori_loop(0, ROWS_MAIN // ZSTEP, zcopy, 0, unroll=False)

    @pl.when(s < NS - 1)
    def _():
        rem = ROWS_MAIN - (ROWS_MAIN // ZSTEP) * ZSTEP
        pltpu.sync_copy(
            rows0.at[pl.ds(0, rem)],
            acc.at[pl.ds(s * ROWS_MAIN + ROWS_MAIN - rem, rem)])

    @pl.when(s == NS - 1)
    def _():
        pltpu.sync_copy(
            rows0.at[pl.ds(0, ZSTEP)],
            acc.at[pl.ds((NS - 1) * ROWS_MAIN + (ROWS_MAIN // ZSTEP) * ZSTEP,
                         ZSTEP)])

    i0.wait()
    i1.wait()
    plsc.subcore_barrier()

    def sidx(k):
        return src_all.at[pl.ds(k * CHUNK, CHUNK)]

    pltpu.async_copy(feat_hbm.at[sidx(0)], rows0, sem0)

    def body(i, _):
        pltpu.async_copy(feat_hbm.at[sidx(2 * i + 1)], rows1, sem1)
        pltpu.make_async_copy(feat_hbm.at[sidx(2 * i)], rows0, sem0).wait()
        pltpu.sync_copy(rows0, acc.at[dst_all.at[2 * i]], add=True)
        pltpu.async_copy(feat_hbm.at[sidx(2 * i + 2)], rows0, sem0)
        pltpu.make_async_copy(feat_hbm.at[sidx(2 * i + 1)], rows1, sem1).wait()
        pltpu.sync_copy(rows1, acc.at[dst_all.at[2 * i + 1]], add=True)
        return 0

    lax.fori_loop(0, N_PAIRS, body, 0, unroll=False)
    pltpu.make_async_copy(feat_hbm.at[sidx(N_ITERS - 1)], rows0, sem0).wait()
    pltpu.sync_copy(rows0, acc.at[dst_all.at[N_ITERS - 1]], add=True)
    plsc.subcore_barrier()

    @pl.when(s < NS - 1)
    def _():
        pltpu.sync_copy(acc.at[pl.ds(s * ROWS_MAIN, ROWS_MAIN)],
                        out_hbm.at[pl.ds(c * N_NODES_C + s * ROWS_MAIN, ROWS_MAIN)])

    @pl.when(s == NS - 1)
    def _():
        pltpu.sync_copy(
            acc.at[pl.ds((NS - 1) * ROWS_MAIN, ROWS_LAST)],
            out_hbm.at[pl.ds(c * N_NODES_C + (NS - 1) * ROWS_MAIN, ROWS_LAST)])


@jax.jit
def _sc_aggregate(feature, e2, e4):
    mesh = plsc.VectorSubcoreMesh(core_axis_name="c", subcore_axis_name="s")
    f = pl.kernel(
        _sc_body,
        out_type=jax.ShapeDtypeStruct((NC * N_NODES_C, D), jnp.float32),
        mesh=mesh,
        scratch_types=[
            pltpu.VMEM((PER_W,), jnp.int32),
            pltpu.VMEM((N_ITERS, CHUNK), jnp.int32),
            pltpu.VMEM((CHUNK, D), jnp.float32),
            pltpu.VMEM((CHUNK, D), jnp.float32),
            pltpu.VMEM_SHARED((N_NODES_C, D), jnp.float32),
            pltpu.SemaphoreType.DMA,
            pltpu.SemaphoreType.DMA,
            pltpu.SemaphoreType.DMA,
        ],
    )
    return f(feature, e2, e4)


def _tc_body(p0_ref, p1_ref, wt_ref, b_ref, o_ref):
    agg = p0_ref[...] + p1_ref[...]
    h = jnp.dot(agg, wt_ref[...], preferred_element_type=jnp.float32)
    o_ref[...] = jnp.maximum(h + b_ref[...], 0.0)


@jax.jit
def _tc_update(partials, Wt, b2):
    blk = 1000
    grid = N_NODES_C // blk
    return pl.pallas_call(
        _tc_body,
        grid=(grid,),
        in_specs=[
            pl.BlockSpec((blk, D), lambda i: (i, 0)),
            pl.BlockSpec((blk, D), lambda i: (i + grid, 0)),
            pl.BlockSpec((D, D), lambda i: (0, 0)),
            pl.BlockSpec((1, D), lambda i: (0, 0)),
        ],
        out_specs=pl.BlockSpec((blk, D), lambda i: (i, 0)),
        out_shape=jax.ShapeDtypeStruct((N_NODES_C, D), jnp.float32),
    )(partials, partials, Wt, b2)


def kernel(feature, edge_index, W, b):
    e2 = edge_index.reshape(2, NW, PER_W)
    e4 = edge_index.reshape(2, NW, N_ITERS, CHUNK)
    partials = _sc_aggregate(feature, e2, e4)
    return _tc_update(partials, W.T, b.reshape(1, D))


# speedup vs baseline: 14.6515x; 1.1541x over previous
import jax
import jax.numpy as jnp
from jax import lax
from jax.experimental import pallas as pl
from jax.experimental.pallas import tpu as pltpu
from jax.experimental.pallas import tpu_sc as plsc

N_NODES_C = 10000
N_EDGES_C = 320000
D = 128

NC = 2
NS = 16
NW = NC * NS

CHUNK = 80
PER_W = N_EDGES_C // NW
N_ITERS = PER_W // CHUNK
N_TRIPLES = (N_ITERS - 5) // 3

ROWS_MAIN = 624
ROWS_LAST = 640
ZSTEP = 80


def _sc_body(feat_hbm, e2_hbm, e4_hbm, out_hbm,
             ibuf, dst_all, rows0, rows1, rows2, acc,
             gsem0, gsem1, gsem2, isem0, isem1, isem2, fsem):
    c = lax.axis_index("c")
    s = lax.axis_index("s")
    wid = s * NC + c

    i1 = pltpu.async_copy(e4_hbm.at[1, wid], dst_all, fsem)

    zeros16 = jnp.zeros((16,), jnp.float32)

    def zrow(i, _):
        for j in range(D // 16):
            rows0[i, pl.ds(j * 16, 16)] = zeros16
        return 0

    lax.fori_loop(0, CHUNK, zrow, 0, unroll=False)

    def zcopy(j, _):
        pltpu.sync_copy(rows0.at[pl.ds(0, ZSTEP)],
                        acc.at[pl.ds(s * ROWS_MAIN + j * ZSTEP, ZSTEP)])
        return 0

    lax.fori_loop(0, ROWS_MAIN // ZSTEP, zcopy, 0, unroll=False)

    @pl.when(s < NS - 1)
    def _():
        rem = ROWS_MAIN - (ROWS_MAIN // ZSTEP) * ZSTEP
        pltpu.sync_copy(
            rows0.at[pl.ds(0, rem)],
            acc.at[pl.ds(s * ROWS_MAIN + ROWS_MAIN - rem, rem)])

    @pl.when(s == NS - 1)
    def _():
        pltpu.sync_copy(
            rows0.at[pl.ds(0, ZSTEP)],
            acc.at[pl.ds((NS - 1) * ROWS_MAIN + (ROWS_MAIN // ZSTEP) * ZSTEP,
                         ZSTEP)])

    i1.wait()

    bufs = (rows0, rows1, rows2)
    gsems = (gsem0, gsem1, gsem2)
    isems = (isem0, isem1, isem2)

    def idxload(k, j):
        pltpu.async_copy(e2_hbm.at[pl.ds(wid * PER_W + k * CHUNK, CHUNK)],
                         ibuf.at[j], isems[j])

    def idxwait(k, j):
        pltpu.make_async_copy(e2_hbm.at[pl.ds(wid * PER_W + k * CHUNK, CHUNK)],
                              ibuf.at[j], isems[j]).wait()

    def gather(j):
        pltpu.async_copy(feat_hbm.at[ibuf.at[j]], bufs[j], gsems[j])

    def gatherwait(j):
        pltpu.make_async_copy(feat_hbm.at[ibuf.at[j]], bufs[j], gsems[j]).wait()

    def scatter(k, j):
        pltpu.sync_copy(bufs[j], acc.at[dst_all.at[k]], add=True)

    for j in range(3):
        idxload(j, j)
    for j in range(3):
        idxwait(j, j)
        gather(j)

    plsc.subcore_barrier()

    def body(g, _):
        cb = 3 * g
        for j in range(3):
            gatherwait(j)
            idxload(cb + j + 3, j)
            scatter(cb + j, j)
            idxwait(cb + j + 3, j)
            gather(j)
        return 0

    lax.fori_loop(0, N_TRIPLES, body, 0, unroll=False)
    tail = 3 * N_TRIPLES
    gatherwait(0)
    idxload(tail + 3, 0)
    scatter(tail, 0)
    idxwait(tail + 3, 0)
    gather(0)
    gatherwait(1)
    idxload(tail + 4, 1)
    scatter(tail + 1, 1)
    idxwait(tail + 4, 1)
    gather(1)
    gatherwait(2)
    scatter(tail + 2, 2)
    gatherwait(0)
    scatter(tail + 3, 0)
    gatherwait(1)
    scatter(tail + 4, 1)
    plsc.subcore_barrier()

    @pl.when(s < NS - 1)
    def _():
        pltpu.sync_copy(acc.at[pl.ds(s * ROWS_MAIN, ROWS_MAIN)],
                        out_hbm.at[pl.ds(c * N_NODES_C + s * ROWS_MAIN, ROWS_MAIN)])

    @pl.when(s == NS - 1)
    def _():
        pltpu.sync_copy(
            acc.at[pl.ds((NS - 1) * ROWS_MAIN, ROWS_LAST)],
            out_hbm.at[pl.ds(c * N_NODES_C + (NS - 1) * ROWS_MAIN, ROWS_LAST)])


@jax.jit
def _sc_aggregate(feature, e2, e4):
    mesh = plsc.VectorSubcoreMesh(core_axis_name="c", subcore_axis_name="s")
    f = pl.kernel(
        _sc_body,
        out_type=jax.ShapeDtypeStruct((NC * N_NODES_C, D), jnp.float32),
        mesh=mesh,
        scratch_types=[
            pltpu.VMEM((3, CHUNK), jnp.int32),
            pltpu.VMEM((N_ITERS, CHUNK), jnp.int32),
            pltpu.VMEM((CHUNK, D), jnp.float32),
            pltpu.VMEM((CHUNK, D), jnp.float32),
            pltpu.VMEM((CHUNK, D), jnp.float32),
            pltpu.VMEM_SHARED((N_NODES_C, D), jnp.float32),
            pltpu.SemaphoreType.DMA,
            pltpu.SemaphoreType.DMA,
            pltpu.SemaphoreType.DMA,
            pltpu.SemaphoreType.DMA,
            pltpu.SemaphoreType.DMA,
            pltpu.SemaphoreType.DMA,
            pltpu.SemaphoreType.DMA,
        ],
    )
    return f(feature, e2, e4)


def _tc_body(p0_ref, p1_ref, wt_ref, b_ref, o_ref):
    agg = p0_ref[...] + p1_ref[...]
    h = jnp.dot(agg, wt_ref[...], preferred_element_type=jnp.float32)
    o_ref[...] = jnp.maximum(h + b_ref[...], 0.0)


@jax.jit
def _tc_update(partials, Wt, b2):
    blk = 1000
    grid = N_NODES_C // blk
    return pl.pallas_call(
        _tc_body,
        grid=(grid,),
        in_specs=[
            pl.BlockSpec((blk, D), lambda i: (i, 0)),
            pl.BlockSpec((blk, D), lambda i: (i + grid, 0)),
            pl.BlockSpec((D, D), lambda i: (0, 0)),
            pl.BlockSpec((1, D), lambda i: (0, 0)),
        ],
        out_specs=pl.BlockSpec((blk, D), lambda i: (i, 0)),
        out_shape=jax.ShapeDtypeStruct((N_NODES_C, D), jnp.float32),
    )(partials, partials, Wt, b2)


def kernel(feature, edge_index, W, b):
    e2 = edge_index.reshape(2 * N_EDGES_C)
    e4 = edge_index.reshape(2, NW, N_ITERS, CHUNK)
    partials = _sc_aggregate(feature, e2, e4)
    return _tc_update(partials, W.T, b.reshape(1, D))


# speedup vs baseline: 15.9426x; 1.0881x over previous
import jax
import jax.numpy as jnp
from jax import lax
from jax.experimental import pallas as pl
from jax.experimental.pallas import tpu as pltpu
from jax.experimental.pallas import tpu_sc as plsc

N_NODES_C = 10000
N_EDGES_C = 320000
D = 128

NC = 2
NS = 16
NW = NC * NS

CHUNK = 80
PER_W = N_EDGES_C // NW
N_ITERS = PER_W // CHUNK
NBUF = 4
N_QUADS = N_ITERS // NBUF - 1

ROWS_MAIN = 624
ROWS_LAST = 640
ZSTEP = 80


def _sc_body(feat_hbm, e1_hbm, out_hbm,
             ibuf, dbuf, rows0, rows1, rows2, rows3, acc,
             gsem0, gsem1, gsem2, gsem3,
             isem0, isem1, isem2, isem3,
             dsem0, dsem1, dsem2, dsem3):
    c = lax.axis_index("c")
    s = lax.axis_index("s")
    wid = s * NC + c

    zeros16 = jnp.zeros((16,), jnp.float32)

    def zrow(i, _):
        for j in range(D // 16):
            rows0[i, pl.ds(j * 16, 16)] = zeros16
        return 0

    lax.fori_loop(0, CHUNK, zrow, 0, unroll=False)

    def zcopy(j, _):
        pltpu.sync_copy(rows0.at[pl.ds(0, ZSTEP)],
                        acc.at[pl.ds(s * ROWS_MAIN + j * ZSTEP, ZSTEP)])
        return 0

    lax.fori_loop(0, ROWS_MAIN // ZSTEP, zcopy, 0, unroll=False)

    @pl.when(s < NS - 1)
    def _():
        rem = ROWS_MAIN - (ROWS_MAIN // ZSTEP) * ZSTEP
        pltpu.sync_copy(
            rows0.at[pl.ds(0, rem)],
            acc.at[pl.ds(s * ROWS_MAIN + ROWS_MAIN - rem, rem)])

    @pl.when(s == NS - 1)
    def _():
        pltpu.sync_copy(
            rows0.at[pl.ds(0, ZSTEP)],
            acc.at[pl.ds((NS - 1) * ROWS_MAIN + (ROWS_MAIN // ZSTEP) * ZSTEP,
                         ZSTEP)])

    bufs = (rows0, rows1, rows2, rows3)
    gsems = (gsem0, gsem1, gsem2, gsem3)
    isems = (isem0, isem1, isem2, isem3)
    dsems = (dsem0, dsem1, dsem2, dsem3)

    def src_ref(k):
        return e1_hbm.at[pl.ds(wid * PER_W + k * CHUNK, CHUNK)]

    def dst_ref(k):
        return e1_hbm.at[pl.ds(N_EDGES_C + wid * PER_W + k * CHUNK, CHUNK)]

    def idxload(k, j):
        pltpu.async_copy(src_ref(k), ibuf.at[j], isems[j])

    def idxwait(k, j):
        pltpu.make_async_copy(src_ref(k), ibuf.at[j], isems[j]).wait()

    def dstload(k, j):
        pltpu.async_copy(dst_ref(k), dbuf.at[j], dsems[j])

    def dstwait(k, j):
        pltpu.make_async_copy(dst_ref(k), dbuf.at[j], dsems[j]).wait()

    def gather(j):
        pltpu.async_copy(feat_hbm.at[ibuf.at[j]], bufs[j], gsems[j])

    def gatherwait(j):
        pltpu.make_async_copy(feat_hbm.at[ibuf.at[j]], bufs[j], gsems[j]).wait()

    def scatter(k, j):
        pltpu.sync_copy(bufs[j], acc.at[dbuf.at[j]], add=True)

    for j in range(NBUF):
        idxload(j, j)
        dstload(j, j)
    for j in range(NBUF):
        idxwait(j, j)
        gather(j)

    plsc.subcore_barrier()

    def body(g, _):
        cb = NBUF * g
        for j in range(NBUF):
            gatherwait(j)
            idxload(cb + j + NBUF, j)
            dstwait(cb + j, j)
            scatter(cb + j, j)
            dstload(cb + j + NBUF, j)
            idxwait(cb + j + NBUF, j)
            gather(j)
        return 0

    lax.fori_loop(0, N_QUADS, body, 0, unroll=False)
    pk = NBUF * N_QUADS
    gatherwait(0)
    idxload(pk + NBUF, 0)
    dstwait(pk, 0)
    scatter(pk, 0)
    dstload(pk + NBUF, 0)
    idxwait(pk + NBUF, 0)
    gather(0)
    for j in range(1, NBUF):
        gatherwait(j)
        dstwait(pk + j, j)
        scatter(pk + j, j)
    gatherwait(0)
    dstwait(pk + NBUF, 0)
    scatter(pk + NBUF, 0)
    plsc.subcore_barrier()

    @pl.when(s < NS - 1)
    def _():
        pltpu.sync_copy(acc.at[pl.ds(s * ROWS_MAIN, ROWS_MAIN)],
                        out_hbm.at[pl.ds(c * N_NODES_C + s * ROWS_MAIN, ROWS_MAIN)])

    @pl.when(s == NS - 1)
    def _():
        pltpu.sync_copy(
            acc.at[pl.ds((NS - 1) * ROWS_MAIN, ROWS_LAST)],
            out_hbm.at[pl.ds(c * N_NODES_C + (NS - 1) * ROWS_MAIN, ROWS_LAST)])


@jax.jit
def _sc_aggregate(feature, e1):
    mesh = plsc.VectorSubcoreMesh(core_axis_name="c", subcore_axis_name="s")
    f = pl.kernel(
        _sc_body,
        out_type=jax.ShapeDtypeStruct((NC * N_NODES_C, D), jnp.float32),
        mesh=mesh,
        scratch_types=(
            [pltpu.VMEM((NBUF, CHUNK), jnp.int32),
             pltpu.VMEM((NBUF, CHUNK), jnp.int32)]
            + [pltpu.VMEM((CHUNK, D), jnp.float32)] * NBUF
            + [pltpu.VMEM_SHARED((N_NODES_C, D), jnp.float32)]
            + [pltpu.SemaphoreType.DMA] * (3 * NBUF)
        ),
    )
    return f(feature, e1)


def _tc_body(p0_ref, p1_ref, wt_ref, b_ref, o_ref):
    agg = p0_ref[...] + p1_ref[...]
    h = jnp.dot(agg, wt_ref[...], preferred_element_type=jnp.float32)
    o_ref[...] = jnp.maximum(h + b_ref[...], 0.0)


@jax.jit
def _tc_update(partials, Wt, b2):
    blk = 1000
    grid = N_NODES_C // blk
    return pl.pallas_call(
        _tc_body,
        grid=(grid,),
        in_specs=[
            pl.BlockSpec((blk, D), lambda i: (i, 0)),
            pl.BlockSpec((blk, D), lambda i: (i + grid, 0)),
            pl.BlockSpec((D, D), lambda i: (0, 0)),
            pl.BlockSpec((1, D), lambda i: (0, 0)),
        ],
        out_specs=pl.BlockSpec((blk, D), lambda i: (i, 0)),
        out_shape=jax.ShapeDtypeStruct((N_NODES_C, D), jnp.float32),
    )(partials, partials, Wt, b2)


def kernel(feature, edge_index, W, b):
    e1 = edge_index.reshape(2 * N_EDGES_C)
    partials = _sc_aggregate(feature, e1)
    return _tc_update(partials, W.T, b.reshape(1, D))


# speedup vs baseline: 16.1492x; 1.0130x over previous
import jax
import jax.numpy as jnp
from jax import lax
from jax.experimental import pallas as pl
from jax.experimental.pallas import tpu as pltpu
from jax.experimental.pallas import tpu_sc as plsc

N_NODES_C = 10000
N_EDGES_C = 320000
D = 128

NC = 2
NS = 16
NW = NC * NS

CH = 128
NCHUNKS = N_EDGES_C // CH
BASE = NCHUNKS // NW
EXTRA = NCHUNKS - BASE * NW
NBUF = 3
N_TURNS = (BASE - NBUF) // NBUF

ROWS_MAIN = 624
ROWS_LAST = 640


def _sc_body(feat_hbm, e_hbm, out_hbm,
             ibuf, rows0, rows1, rows2, acc,
             gsem0, gsem1, gsem2, isem0, isem1, isem2):
    c = lax.axis_index("c")
    s = lax.axis_index("s")
    wid = s * NC + c
    start = wid * BASE + jnp.minimum(wid, EXTRA)
    has_extra = wid < EXTRA

    zeros16 = jnp.zeros((16,), jnp.float32)

    def zrow(i, _):
        for j in range(D // 16):
            rows0[i, pl.ds(j * 16, 16)] = zeros16
        return 0

    lax.fori_loop(0, CH, zrow, 0, unroll=False)

    def zcopy(j, _):
        pltpu.sync_copy(rows0, acc.at[pl.ds(s * ROWS_MAIN + j * CH, CH)])
        return 0

    lax.fori_loop(0, ROWS_MAIN // CH, zcopy, 0, unroll=False)
    zbase = (ROWS_MAIN // CH) * CH

    @pl.when(s < NS - 1)
    def _():
        pltpu.sync_copy(rows0.at[pl.ds(0, ROWS_MAIN - zbase)],
                        acc.at[pl.ds(s * ROWS_MAIN + zbase, ROWS_MAIN - zbase)])

    @pl.when(s == NS - 1)
    def _():
        pltpu.sync_copy(rows0,
                        acc.at[pl.ds((NS - 1) * ROWS_MAIN + zbase, CH)])

    bufs = (rows0, rows1, rows2)
    gsems = (gsem0, gsem1, gsem2)
    isems = (isem0, isem1, isem2)

    def eref(m):
        return e_hbm.at[:, pl.ds(m * CH, CH)]

    def idxload(m, j):
        pltpu.async_copy(eref(m), ibuf.at[j], isems[j])

    def idxwait(m, j):
        pltpu.make_async_copy(eref(m), ibuf.at[j], isems[j]).wait()

    def gather(j):
        pltpu.async_copy(feat_hbm.at[ibuf.at[j, 0]], bufs[j], gsems[j])

    def gatherwait(j):
        pltpu.make_async_copy(feat_hbm.at[ibuf.at[j, 0]], bufs[j], gsems[j]).wait()

    def scatter(j):
        pltpu.sync_copy(bufs[j], acc.at[ibuf.at[j, 1]], add=True)

    for j in range(NBUF):
        idxload(start + j, j)
    for j in range(NBUF):
        idxwait(start + j, j)
        gather(j)

    plsc.subcore_barrier()

    def body(g, _):
        lb = NBUF * g
        for j in range(NBUF):
            gatherwait(j)
            idxload(start + lb + j + NBUF, j)
            scatter(j)
            idxwait(start + lb + j + NBUF, j)
            gather(j)
        return 0

    lax.fori_loop(0, N_TURNS, body, 0, unroll=False)

    lt = BASE - NBUF

    gatherwait(0)

    @pl.when(has_extra)
    def _():
        idxload(start + BASE, 0)

    scatter(0)

    @pl.when(has_extra)
    def _():
        idxwait(start + BASE, 0)
        gather(0)

    for j in range(1, NBUF):
        gatherwait(j)
        scatter(j)

    @pl.when(has_extra)
    def _():
        gatherwait(0)
        scatter(0)

    plsc.subcore_barrier()

    @pl.when(s < NS - 1)
    def _():
        pltpu.sync_copy(acc.at[pl.ds(s * ROWS_MAIN, ROWS_MAIN)],
                        out_hbm.at[pl.ds(c * N_NODES_C + s * ROWS_MAIN, ROWS_MAIN)])

    @pl.when(s == NS - 1)
    def _():
        pltpu.sync_copy(
            acc.at[pl.ds((NS - 1) * ROWS_MAIN, ROWS_LAST)],
            out_hbm.at[pl.ds(c * N_NODES_C + (NS - 1) * ROWS_MAIN, ROWS_LAST)])


@jax.jit
def _sc_aggregate(feature, edge_index):
    mesh = plsc.VectorSubcoreMesh(core_axis_name="c", subcore_axis_name="s")
    f = pl.kernel(
        _sc_body,
        out_type=jax.ShapeDtypeStruct((NC * N_NODES_C, D), jnp.float32),
        mesh=mesh,
        scratch_types=(
            [pltpu.VMEM((NBUF, 2, CH), jnp.int32)]
            + [pltpu.VMEM((CH, D), jnp.float32)] * NBUF
            + [pltpu.VMEM_SHARED((N_NODES_C, D), jnp.float32)]
            + [pltpu.SemaphoreType.DMA] * (2 * NBUF)
        ),
    )
    return f(feature, edge_index)


def _tc_body(p0_ref, p1_ref, wt_ref, b_ref, o_ref):
    agg = p0_ref[...] + p1_ref[...]
    h = jnp.dot(agg, wt_ref[...], preferred_element_type=jnp.float32)
    o_ref[...] = jnp.maximum(h + b_ref[...], 0.0)


@jax.jit
def _tc_update(partials, Wt, b2):
    blk = 1000
    grid = N_NODES_C // blk
    return pl.pallas_call(
        _tc_body,
        grid=(grid,),
        in_specs=[
            pl.BlockSpec((blk, D), lambda i: (i, 0)),
            pl.BlockSpec((blk, D), lambda i: (i + grid, 0)),
            pl.BlockSpec((D, D), lambda i: (0, 0)),
            pl.BlockSpec((1, D), lambda i: (0, 0)),
        ],
        out_specs=pl.BlockSpec((blk, D), lambda i: (i, 0)),
        out_shape=jax.ShapeDtypeStruct((N_NODES_C, D), jnp.float32),
    )(partials, partials, Wt, b2)


def kernel(feature, edge_index, W, b):
    partials = _sc_aggregate(feature, edge_index)
    return _tc_update(partials, W.T, b.reshape(1, D))


# speedup vs baseline: 16.6552x; 1.0313x over previous
import jax
import jax.numpy as jnp
from jax import lax
from jax.experimental import pallas as pl
from jax.experimental.pallas import tpu as pltpu
from jax.experimental.pallas import tpu_sc as plsc

N_NODES_C = 10000
N_EDGES_C = 320000
D = 128

NC = 2
NS = 16
NW = NC * NS

CH = 128
NCHUNKS = N_EDGES_C // CH
BASE = NCHUNKS // NW
EXTRA = NCHUNKS - BASE * NW
NBUF = 3
N_TURNS = (BASE - NBUF) // NBUF

ROWS_MAIN = 624
ROWS_LAST = 640


def _sc_body(feat_hbm, e_hbm, out_hbm,
             ibuf, rows0, rows1, rows2, acc,
             gsem0, gsem1, gsem2, isem0, isem1, isem2):
    c = lax.axis_index("c")
    s = lax.axis_index("s")
    wid = s * NC + c
    start = wid * BASE + jnp.minimum(wid, EXTRA)
    has_extra = wid < EXTRA

    zeros16 = jnp.zeros((16,), jnp.float32)

    def zrow(i, _):
        for j in range(D // 16):
            rows0[i, pl.ds(j * 16, 16)] = zeros16
        return 0

    lax.fori_loop(0, CH, zrow, 0, unroll=False)

    def zcopy(j, _):
        pltpu.sync_copy(rows0, acc.at[pl.ds(s * ROWS_MAIN + j * CH, CH)])
        return 0

    lax.fori_loop(0, ROWS_MAIN // CH, zcopy, 0, unroll=False)
    zbase = (ROWS_MAIN // CH) * CH

    @pl.when(s < NS - 1)
    def _():
        pltpu.sync_copy(rows0.at[pl.ds(0, ROWS_MAIN - zbase)],
                        acc.at[pl.ds(s * ROWS_MAIN + zbase, ROWS_MAIN - zbase)])

    @pl.when(s == NS - 1)
    def _():
        pltpu.sync_copy(rows0,
                        acc.at[pl.ds((NS - 1) * ROWS_MAIN + zbase, CH)])

    bufs = (rows0, rows1, rows2)
    gsems = (gsem0, gsem1, gsem2)
    isems = (isem0, isem1, isem2)

    def eref(m):
        return e_hbm.at[:, pl.ds(m * CH, CH)]

    def idxload(m, j):
        pltpu.async_copy(eref(m), ibuf.at[j], isems[j])

    def idxwait(m, j):
        pltpu.make_async_copy(eref(m), ibuf.at[j], isems[j]).wait()

    def gather(j):
        pltpu.async_copy(feat_hbm.at[ibuf.at[j, 0]], bufs[j], gsems[j])

    def gatherwait(j):
        pltpu.make_async_copy(feat_hbm.at[ibuf.at[j, 0]], bufs[j], gsems[j]).wait()

    def scatter(j):
        pltpu.sync_copy(bufs[j], acc.at[ibuf.at[j, 1]], add=True)

    for j in range(NBUF):
        idxload(start + j, j)
    for j in range(NBUF):
        idxwait(start + j, j)
        gather(j)

    plsc.subcore_barrier()

    def body(g, _):
        lb = NBUF * g
        for j in range(NBUF):
            gatherwait(j)
            idxload(start + lb + j + NBUF, j)
            scatter(j)
            idxwait(start + lb + j + NBUF, j)
            gather(j)
        return 0

    lax.fori_loop(0, N_TURNS, body, 0, unroll=False)

    lt = BASE - NBUF

    gatherwait(0)

    @pl.when(has_extra)
    def _():
        idxload(start + BASE, 0)

    scatter(0)

    @pl.when(has_extra)
    def _():
        idxwait(start + BASE, 0)
        gather(0)

    for j in range(1, NBUF):
        gatherwait(j)
        scatter(j)

    @pl.when(has_extra)
    def _():
        gatherwait(0)
        scatter(0)

    plsc.subcore_barrier()

    @pl.when(s < NS - 1)
    def _():
        pltpu.sync_copy(acc.at[pl.ds(s * ROWS_MAIN, ROWS_MAIN)],
                        out_hbm.at[pl.ds(c * N_NODES_C + s * ROWS_MAIN, ROWS_MAIN)])

    @pl.when(s == NS - 1)
    def _():
        pltpu.sync_copy(
            acc.at[pl.ds((NS - 1) * ROWS_MAIN, ROWS_LAST)],
            out_hbm.at[pl.ds(c * N_NODES_C + (NS - 1) * ROWS_MAIN, ROWS_LAST)])


@jax.jit
def _sc_aggregate(feature, edge_index):
    mesh = plsc.VectorSubcoreMesh(core_axis_name="c", subcore_axis_name="s")
    f = pl.kernel(
        _sc_body,
        out_type=jax.ShapeDtypeStruct((NC * N_NODES_C, D), jnp.float32),
        mesh=mesh,
        scratch_types=(
            [pltpu.VMEM((NBUF, 2, CH), jnp.int32)]
            + [pltpu.VMEM((CH, D), jnp.float32)] * NBUF
            + [pltpu.VMEM_SHARED((N_NODES_C, D), jnp.float32)]
            + [pltpu.SemaphoreType.DMA] * (2 * NBUF)
        ),
    )
    return f(feature, edge_index)


def _tc_body(p0_ref, p1_ref, wt_ref, b_ref, o_ref):
    agg = p0_ref[...] + p1_ref[...]
    h = jnp.dot(agg, wt_ref[...], preferred_element_type=jnp.float32)
    o_ref[...] = jnp.maximum(h + b_ref[...], 0.0)


@jax.jit
def _tc_update(partials, Wt, b2):
    blk = 2000
    grid = N_NODES_C // blk
    return pl.pallas_call(
        _tc_body,
        grid=(grid,),
        in_specs=[
            pl.BlockSpec((blk, D), lambda i: (i, 0)),
            pl.BlockSpec((blk, D), lambda i: (i + grid, 0)),
            pl.BlockSpec((D, D), lambda i: (0, 0)),
            pl.BlockSpec((1, D), lambda i: (0, 0)),
        ],
        out_specs=pl.BlockSpec((blk, D), lambda i: (i, 0)),
        out_shape=jax.ShapeDtypeStruct((N_NODES_C, D), jnp.float32),
    )(partials, partials, Wt, b2)


def kernel(feature, edge_index, W, b):
    partials = _sc_aggregate(feature, edge_index)
    return _tc_update(partials, W.T, b.reshape(1, D))


# speedup vs baseline: 16.8973x; 1.0145x over previous
import jax
import jax.numpy as jnp
from jax import lax
from jax.experimental import pallas as pl
from jax.experimental.pallas import tpu as pltpu
from jax.experimental.pallas import tpu_sc as plsc

N_NODES_C = 10000
N_EDGES_C = 320000
D = 128

NC = 2
NS = 16
NW = NC * NS

CH = 128
NCHUNKS = N_EDGES_C // CH
BASE = NCHUNKS // NW
EXTRA = NCHUNKS - BASE * NW
NBUF = 3
N_TURNS = (BASE - NBUF) // NBUF

ROWS_MAIN = 624
ROWS_LAST = 640


def _sc_body(feat_hbm, e_hbm, out_hbm,
             ibuf, rows0, rows1, rows2, acc,
             gsem0, gsem1, gsem2, isem0, isem1, isem2):
    c = lax.axis_index("c")
    s = lax.axis_index("s")
    wid = s * NC + c
    start = wid * BASE + jnp.minimum(wid, EXTRA)
    has_extra = wid < EXTRA

    zeros16 = jnp.zeros((16,), jnp.float32)

    def zrow(i, _):
        for j in range(D // 16):
            rows0[i, pl.ds(j * 16, 16)] = zeros16
        return 0

    lax.fori_loop(0, CH, zrow, 0, unroll=False)

    def zcopy(j, _):
        pltpu.sync_copy(rows0, acc.at[pl.ds(s * ROWS_MAIN + j * CH, CH)])
        return 0

    lax.fori_loop(0, ROWS_MAIN // CH, zcopy, 0, unroll=False)
    zbase = (ROWS_MAIN // CH) * CH

    @pl.when(s < NS - 1)
    def _():
        pltpu.sync_copy(rows0.at[pl.ds(0, ROWS_MAIN - zbase)],
                        acc.at[pl.ds(s * ROWS_MAIN + zbase, ROWS_MAIN - zbase)])

    @pl.when(s == NS - 1)
    def _():
        pltpu.sync_copy(rows0,
                        acc.at[pl.ds((NS - 1) * ROWS_MAIN + zbase, CH)])

    bufs = (rows0, rows1, rows2)
    gsems = (gsem0, gsem1, gsem2)
    isems = (isem0, isem1, isem2)

    def eref(m):
        return e_hbm.at[:, pl.ds(m * CH, CH)]

    def idxload(m, j):
        pltpu.async_copy(eref(m), ibuf.at[j], isems[j])

    def idxwait(m, j):
        pltpu.make_async_copy(eref(m), ibuf.at[j], isems[j]).wait()

    def gather(j):
        pltpu.async_copy(feat_hbm.at[ibuf.at[j, 0]], bufs[j], gsems[j])

    def gatherwait(j):
        pltpu.make_async_copy(feat_hbm.at[ibuf.at[j, 0]], bufs[j], gsems[j]).wait()

    def scatter(j):
        pltpu.sync_copy(bufs[j], acc.at[ibuf.at[j, 1]], add=True)

    for j in range(NBUF):
        idxload(start + j, j)
    for j in range(NBUF):
        idxwait(start + j, j)
        gather(j)

    plsc.subcore_barrier()

    def body(g, _):
        lb = NBUF * g
        for j in range(NBUF):
            gatherwait(j)
            idxload(start + lb + j + NBUF, j)
            scatter(j)
            idxwait(start + lb + j + NBUF, j)
            gather(j)
        return 0

    lax.fori_loop(0, N_TURNS, body, 0, unroll=False)

    lt = BASE - NBUF

    gatherwait(0)

    @pl.when(has_extra)
    def _():
        idxload(start + BASE, 0)

    scatter(0)

    @pl.when(has_extra)
    def _():
        idxwait(start + BASE, 0)
        gather(0)

    for j in range(1, NBUF):
        gatherwait(j)
        scatter(j)

    @pl.when(has_extra)
    def _():
        gatherwait(0)
        scatter(0)

    plsc.subcore_barrier()

    @pl.when(s < NS - 1)
    def _():
        pltpu.sync_copy(acc.at[pl.ds(s * ROWS_MAIN, ROWS_MAIN)],
                        out_hbm.at[pl.ds(c * N_NODES_C + s * ROWS_MAIN, ROWS_MAIN)])

    @pl.when(s == NS - 1)
    def _():
        pltpu.sync_copy(
            acc.at[pl.ds((NS - 1) * ROWS_MAIN, ROWS_LAST)],
            out_hbm.at[pl.ds(c * N_NODES_C + (NS - 1) * ROWS_MAIN, ROWS_LAST)])


@jax.jit
def _sc_aggregate(feature, edge_index):
    mesh = plsc.VectorSubcoreMesh(core_axis_name="c", subcore_axis_name="s")
    f = pl.kernel(
        _sc_body,
        out_type=jax.ShapeDtypeStruct((NC * N_NODES_C, D), jnp.float32),
        mesh=mesh,
        scratch_types=(
            [pltpu.VMEM((NBUF, 2, CH), jnp.int32)]
            + [pltpu.VMEM((CH, D), jnp.float32)] * NBUF
            + [pltpu.VMEM_SHARED((N_NODES_C, D), jnp.float32)]
            + [pltpu.SemaphoreType.DMA] * (2 * NBUF)
        ),
    )
    return f(feature, edge_index)


def _tc_body(p0_ref, p1_ref, wt_ref, b_ref, o_ref):
    agg = p0_ref[...] + p1_ref[...]
    h = jnp.dot(agg, wt_ref[...], preferred_element_type=jnp.float32)
    o_ref[...] = jnp.maximum(h + b_ref[...], 0.0)


@jax.jit
def _tc_update(partials, Wt, b2):
    blk = 5000
    grid = N_NODES_C // blk
    return pl.pallas_call(
        _tc_body,
        grid=(grid,),
        in_specs=[
            pl.BlockSpec((blk, D), lambda i: (i, 0)),
            pl.BlockSpec((blk, D), lambda i: (i + grid, 0)),
            pl.BlockSpec((D, D), lambda i: (0, 0)),
            pl.BlockSpec((1, D), lambda i: (0, 0)),
        ],
        out_specs=pl.BlockSpec((blk, D), lambda i: (i, 0)),
        out_shape=jax.ShapeDtypeStruct((N_NODES_C, D), jnp.float32),
    )(partials, partials, Wt, b2)


def kernel(feature, edge_index, W, b):
    partials = _sc_aggregate(feature, edge_index)
    return _tc_update(partials, W.T, b.reshape(1, D))
